# baseline (speedup 1.0000x reference)
import functools

import jax
import jax.numpy as jnp
from jax import lax
from jax.experimental import pallas as pl
from jax.experimental.pallas import tpu as pltpu
from jax.experimental.pallas import tpu_sc as plsc

NUM_CORES = 2
NUM_SUBCORES = 16
NUM_WORKERS = NUM_CORES * NUM_SUBCORES
CW = 4
LANES = 128
CHUNK_W = CW * LANES


def _full(v):
    return jnp.full((16,), v, jnp.int32)


@functools.lru_cache(maxsize=None)
def _make_gather(B, D, V):
    n_tc = (V + LANES - 1) // LANES
    rng = (n_tc + NUM_WORKERS - 1) // NUM_WORKERS
    n_ch = (rng + CW - 1) // CW
    assert n_ch % 2 == 0
    phys_lanes = n_tc * LANES
    last_start = phys_lanes - CHUNK_W
    wl_cap = B + n_ch * 16
    n_fill = (wl_cap + 15) // 16
    n_groups = B // 16
    mesh = plsc.VectorSubcoreMesh(core_axis_name="c", subcore_axis_name="s")

    @functools.partial(
        pl.kernel,
        mesh=mesh,
        out_type=jax.ShapeDtypeStruct((B, D), jnp.float32),
        scratch_types=[
            pltpu.VMEM((B,), jnp.int32),
            pltpu.VMEM((wl_cap,), jnp.int32),
            pltpu.VMEM((wl_cap,), jnp.int32),
            pltpu.VMEM((16,), jnp.int32),
            pltpu.VMEM((64,), jnp.int32),
            pltpu.VMEM((64,), jnp.int32),
            pltpu.VMEM((64,), jnp.int32),
            pltpu.VMEM((2, D, CHUNK_W), jnp.float32),
            pltpu.VMEM((2, 16, D), jnp.float32),
            pltpu.HBM((D, CHUNK_W), jnp.float32),
            pltpu.HBM((16, D), jnp.float32),
            pltpu.SemaphoreType.DMA,
            pltpu.SemaphoreType.DMA,
            pltpu.SemaphoreType.DMA,
            pltpu.SemaphoreType.DMA,
        ],
        compiler_params=pltpu.CompilerParams(
            use_tc_tiling_on_sc=True,
            needs_layout_passes=False,
            disable_bounds_checks=True,
        ),
    )
    def gather_kernel(
        tab_hbm, idx_hbm, out_hbm,
        idx_v, wl, wl2, tmp16, hist, bases, cursors, cbuf, rowbuf,
        dummy, trash, s0, s1, sr0, sr1,
    ):
        wid = lax.axis_index("s") * NUM_CORES + lax.axis_index("c")
        lo = wid * rng
        lane = lax.iota(jnp.int32, 16)
        pltpu.sync_copy(idx_hbm, idx_v)

        def chunk_start(c):
            return jnp.minimum((lo + c * CW) * LANES, last_start)

        def issue_chunk(c, b, sem):
            start = pl.multiple_of(chunk_start(c), LANES)
            pltpu.async_copy(
                tab_hbm.at[:, pl.ds(start, CHUNK_W)], cbuf.at[b], sem
            )

        issue_chunk(0, 0, s0)
        issue_chunk(1, 1, s1)

        def fill_body(k, carry):
            wl[pl.ds(k * 16, 16)] = _full(-1)
            wl2[pl.ds(k * 16, 16)] = _full(-1)
            return carry

        lax.fori_loop(0, n_fill, fill_body, 0)
        hist[pl.ds(0, 16)] = _full(0)
        hist[pl.ds(16, 16)] = _full(0)
        hist[pl.ds(32, 16)] = _full(0)
        hist[pl.ds(48, 16)] = _full(0)

        def scan_body(g, tot):
            v = idx_v[pl.ds(g * 16, 16)]
            ti = lax.shift_right_logical(v, 7)
            trel = ti - lo
            mine = jnp.logical_and(trel >= 0, trel < rng)
            pc = plsc.all_reduce_population_count(mine)

            @pl.when(pc[0] > 0)
            def _():
                li = jnp.bitwise_and(v, LANES - 1)
                pos = lane + g * 16
                entry = li + pos * 128 + trel * (1 << 21)
                m01 = jnp.where(mine, 1, 0)
                rank = plsc.cumsum(m01) - m01
                plsc.store_scatter(wl, [tot + rank], entry, mask=mine)

            return tot + pc

        totv = lax.fori_loop(0, n_groups, scan_body, jnp.zeros((16,), jnp.int32))
        total = jnp.max(totv)
        ng_wl = lax.shift_right_logical(total + 15, 4)

        def segments(se):
            valid = se >= 0
            chunk = lax.shift_right_logical(se, 23)
            chunk = jnp.where(valid, chunk, _full(-1))
            tmp16[pl.ds(0, 16)] = chunk
            prev = plsc.load_gather(tmp16, [jnp.maximum(lane - 1, 0)])
            nxt = plsc.load_gather(tmp16, [jnp.minimum(lane + 1, 15)])
            boundary = jnp.logical_or(chunk != prev, lane == 0)
            lastseg = jnp.logical_or(chunk != nxt, lane == 15)
            segfirst = plsc.cummax(jnp.where(boundary, lane, 0))
            rank = lane - segfirst
            return valid, chunk, rank, lastseg

        def b1_body(g, carry):
            ev = wl[pl.ds(g * 16, 16)]
            se, _ = plsc.sort_key_val(ev, ev)
            wl[pl.ds(g * 16, 16)] = se
            valid, chunk, rank, lastseg = segments(se)
            upd = jnp.logical_and(valid, lastseg)
            cnt = plsc.load_gather(hist, [jnp.maximum(chunk, 0)]) + rank + 1
            plsc.store_scatter(hist, [chunk], cnt, mask=upd)
            return carry

        lax.fori_loop(0, ng_wl, b1_body, 0)

        carry = jnp.zeros((16,), jnp.int32)
        for r in range(4):
            h = hist[pl.ds(r * 16, 16)]
            padded = jnp.bitwise_and(h + 15, ~15)
            incl = plsc.cumsum(padded)
            base = incl - padded + carry
            bases[pl.ds(r * 16, 16)] = base
            cursors[pl.ds(r * 16, 16)] = base
            carry = carry + jnp.max(incl)

        def b2_body(g, carry2):
            se = wl[pl.ds(g * 16, 16)]
            valid, chunk, rank, lastseg = segments(se)
            cur = plsc.load_gather(cursors, [jnp.maximum(chunk, 0)])
            slot = cur + rank
            plsc.store_scatter(wl2, [slot], se, mask=valid)
            upd = jnp.logical_and(valid, lastseg)
            plsc.store_scatter(cursors, [chunk], slot + 1, mask=upd)
            return carry2

        lax.fori_loop(0, ng_wl, b2_body, 0)

        srs = (sr0, sr1)
        pltpu.async_copy(rowbuf.at[0], trash, sr0)
        pltpu.async_copy(rowbuf.at[1], trash, sr1)

        def process_chunk(c, b):
            base = jnp.max(plsc.load_gather(bases, [_full(c)]))
            cnt = jnp.max(plsc.load_gather(hist, [_full(c)]))
            ng = lax.shift_right_logical(cnt + 15, 4)
            startv = _full(chunk_start(c))

            def do_group(k, r):
                off = base + k * 16
                ev = wl2[pl.ds(off, 16)]
                m = ev >= 0
                li = jnp.bitwise_and(ev, LANES - 1)
                pos = jnp.bitwise_and(
                    lax.shift_right_logical(ev, 7), (1 << 14) - 1
                )
                trel = lax.shift_right_logical(ev, 21)
                loc = (lo + trel) * LANES + li - startv
                pltpu.make_async_copy(trash, rowbuf.at[r], srs[r]).wait()
                for j in range(D):
                    val = plsc.load_gather(
                        cbuf, [_full(b), _full(j), loc], mask=m
                    )
                    plsc.store_scatter(
                        rowbuf, [_full(r), lane, _full(j)], val, mask=m
                    )
                for l in range(16):
                    dst_ok = out_hbm.at[pos[l]]

                    @pl.when(ev[l] >= 0)
                    def _():
                        pltpu.async_copy(rowbuf.at[r, l], dst_ok, srs[r])

                    @pl.when(ev[l] < 0)
                    def _():
                        pltpu.async_copy(rowbuf.at[r, l], trash.at[l], srs[r])

            def grp_body(k2, carry3):
                for r in range(2):
                    k = k2 * 2 + r

                    @pl.when(k < ng)
                    def _():
                        do_group(k, r)

                return carry3

            lax.fori_loop(0, lax.shift_right_logical(ng + 1, 1), grp_body, 0)

        def stream_body(j, carry4):
            c0 = j * 2
            c1 = c0 + 1
            pltpu.make_async_copy(dummy, cbuf.at[0], s0).wait()
            process_chunk(c0, 0)

            @pl.when(c0 + 2 < n_ch)
            def _():
                issue_chunk(c0 + 2, 0, s0)

            pltpu.make_async_copy(dummy, cbuf.at[1], s1).wait()
            process_chunk(c1, 1)

            @pl.when(c1 + 2 < n_ch)
            def _():
                issue_chunk(c1 + 2, 1, s1)

            return carry4

        lax.fori_loop(0, n_ch // 2, stream_body, 0)
        pltpu.make_async_copy(trash, rowbuf.at[0], sr0).wait()
        pltpu.make_async_copy(trash, rowbuf.at[1], sr1).wait()

    return gather_kernel


def kernel(x, embed_weight):
    (B,) = x.shape
    V, D = embed_weight.shape
    tab_t = embed_weight.T
    idx = x.astype(jnp.int32)
    out = _make_gather(B, D, V)(tab_t, idx)
    return out[None]


# speedup vs baseline: 1.0362x; 1.0362x over previous
import functools

import jax
import jax.numpy as jnp
from jax import lax
from jax.experimental import pallas as pl
from jax.experimental.pallas import tpu as pltpu
from jax.experimental.pallas import tpu_sc as plsc

NUM_CORES = 2
NUM_SUBCORES = 16
NUM_WORKERS = NUM_CORES * NUM_SUBCORES
CW = 4
LANES = 128
CHUNK_W = CW * LANES


def _full(v):
    return jnp.full((16,), v, jnp.int32)


@functools.lru_cache(maxsize=None)
def _make_gather(B, D, V):
    n_tc = (V + LANES - 1) // LANES
    rng = (n_tc + NUM_WORKERS - 1) // NUM_WORKERS
    n_ch = (rng + CW - 1) // CW
    assert n_ch % 2 == 0
    phys_lanes = n_tc * LANES
    last_start = phys_lanes - CHUNK_W
    wl_cap = B + n_ch * 16
    n_fill = (wl_cap + 15) // 16
    n_groups = B // 16
    mesh = plsc.VectorSubcoreMesh(core_axis_name="c", subcore_axis_name="s")

    @functools.partial(
        pl.kernel,
        mesh=mesh,
        out_type=jax.ShapeDtypeStruct((B, D), jnp.float32),
        scratch_types=[
            pltpu.VMEM((B,), jnp.int32),
            pltpu.VMEM((wl_cap,), jnp.int32),
            pltpu.VMEM((wl_cap,), jnp.int32),
            pltpu.VMEM((16,), jnp.int32),
            pltpu.VMEM((64,), jnp.int32),
            pltpu.VMEM((64,), jnp.int32),
            pltpu.VMEM((64,), jnp.int32),
            pltpu.VMEM((2, D, CHUNK_W), jnp.float32),
            pltpu.VMEM((16, D), jnp.float32),
            pltpu.HBM((D, CHUNK_W), jnp.float32),
            pltpu.HBM((16, D), jnp.float32),
            pltpu.SemaphoreType.DMA,
            pltpu.SemaphoreType.DMA,
            pltpu.SemaphoreType.DMA,
        ],
        compiler_params=pltpu.CompilerParams(
            use_tc_tiling_on_sc=True,
            needs_layout_passes=False,
            disable_bounds_checks=True,
        ),
    )
    def gather_kernel(
        tab_hbm, idx_hbm, out_hbm,
        idx_v, wl, wl2, tmp16, hist, bases, cursors, cbuf, rowbuf,
        dummy, trash, s0, s1, sr,
    ):
        wid = lax.axis_index("s") * NUM_CORES + lax.axis_index("c")
        lo = wid * rng
        lane = lax.iota(jnp.int32, 16)
        pltpu.sync_copy(idx_hbm, idx_v)

        def chunk_start(c):
            return jnp.minimum((lo + c * CW) * LANES, last_start)

        def issue_chunk(c, b, sem):
            start = pl.multiple_of(chunk_start(c), LANES)
            pltpu.async_copy(
                tab_hbm.at[:, pl.ds(start, CHUNK_W)], cbuf.at[b], sem
            )

        issue_chunk(0, 0, s0)
        issue_chunk(1, 1, s1)

        hist[pl.ds(0, 16)] = _full(0)
        hist[pl.ds(16, 16)] = _full(0)
        hist[pl.ds(32, 16)] = _full(0)
        hist[pl.ds(48, 16)] = _full(0)

        def scan_body(g, tot):
            for q in range(4):
                gq = g * 4 + q
                v = idx_v[pl.ds(gq * 16, 16)]
                ti = lax.shift_right_logical(v, 7)
                trel = ti - lo
                mine = jnp.logical_and(trel >= 0, trel < rng)
                pc = plsc.all_reduce_population_count(mine)
                t = tot

                @pl.when(pc[0] > 0)
                def _():
                    li = jnp.bitwise_and(v, LANES - 1)
                    pos = lane + gq * 16
                    entry = li + pos * 128 + trel * (1 << 21)
                    m01 = jnp.where(mine, 1, 0)
                    rank = plsc.cumsum(m01) - m01
                    plsc.store_scatter(wl, [t + rank], entry, mask=mine)

                tot = tot + pc
            return tot

        totv = lax.fori_loop(
            0, n_groups // 4, scan_body, jnp.zeros((16,), jnp.int32)
        )
        total = jnp.max(totv)
        ng_wl = lax.shift_right_logical(total + 15, 4)
        plsc.store_scatter(wl, [totv + lane], _full(-1))

        def segments(se):
            valid = se >= 0
            chunk = lax.shift_right_logical(se, 23)
            chunk = jnp.where(valid, chunk, _full(-1))
            tmp16[pl.ds(0, 16)] = chunk
            prev = plsc.load_gather(tmp16, [jnp.maximum(lane - 1, 0)])
            nxt = plsc.load_gather(tmp16, [jnp.minimum(lane + 1, 15)])
            boundary = jnp.logical_or(chunk != prev, lane == 0)
            lastseg = jnp.logical_or(chunk != nxt, lane == 15)
            segfirst = plsc.cummax(jnp.where(boundary, lane, 0))
            rank = lane - segfirst
            return valid, chunk, rank, lastseg

        def b1_body(g, carry):
            ev = wl[pl.ds(g * 16, 16)]
            se, _ = plsc.sort_key_val(ev, ev)
            wl[pl.ds(g * 16, 16)] = se
            valid, chunk, rank, lastseg = segments(se)
            upd = jnp.logical_and(valid, lastseg)
            cnt = plsc.load_gather(hist, [jnp.maximum(chunk, 0)]) + rank + 1
            plsc.store_scatter(hist, [chunk], cnt, mask=upd)
            return carry

        lax.fori_loop(0, ng_wl, b1_body, 0)

        carry = jnp.zeros((16,), jnp.int32)
        for r in range(4):
            h = hist[pl.ds(r * 16, 16)]
            padded = jnp.bitwise_and(h + 15, ~15)
            incl = plsc.cumsum(padded)
            base = incl - padded + carry
            bases[pl.ds(r * 16, 16)] = base
            cursors[pl.ds(r * 16, 16)] = base
            carry = carry + jnp.max(incl)

        def fill2_body(k, c2):
            wl2[pl.ds(k * 16, 16)] = _full(-1)
            return c2

        lax.fori_loop(
            0, lax.shift_right_logical(jnp.max(carry), 4), fill2_body, 0
        )

        def b2_body(g, carry2):
            se = wl[pl.ds(g * 16, 16)]
            valid, chunk, rank, lastseg = segments(se)
            cur = plsc.load_gather(cursors, [jnp.maximum(chunk, 0)])
            slot = cur + rank
            plsc.store_scatter(wl2, [slot], se, mask=valid)
            upd = jnp.logical_and(valid, lastseg)
            plsc.store_scatter(cursors, [chunk], slot + 1, mask=upd)
            return carry2

        lax.fori_loop(0, ng_wl, b2_body, 0)

        pltpu.async_copy(rowbuf, trash, sr)

        def process_chunk(c, b):
            base = jnp.max(plsc.load_gather(bases, [_full(c)]))
            cnt = jnp.max(plsc.load_gather(hist, [_full(c)]))
            ng = lax.shift_right_logical(cnt + 15, 4)
            startv = _full(chunk_start(c))

            def grp_body(k, carry3):
                off = base + k * 16
                ev = wl2[pl.ds(off, 16)]
                m = ev >= 0
                li = jnp.bitwise_and(ev, LANES - 1)
                pos = jnp.bitwise_and(
                    lax.shift_right_logical(ev, 7), (1 << 14) - 1
                )
                trel = lax.shift_right_logical(ev, 21)
                loc = (lo + trel) * LANES + li - startv
                pltpu.make_async_copy(trash, rowbuf, sr).wait()
                for j in range(D):
                    val = plsc.load_gather(
                        cbuf, [_full(b), _full(j), loc], mask=m
                    )
                    plsc.store_scatter(rowbuf, [lane, _full(j)], val, mask=m)
                for l in range(16):
                    dst_ok = out_hbm.at[pos[l]]

                    @pl.when(ev[l] >= 0)
                    def _():
                        pltpu.async_copy(rowbuf.at[l], dst_ok, sr)

                    @pl.when(ev[l] < 0)
                    def _():
                        pltpu.async_copy(rowbuf.at[l], trash.at[l], sr)

                return carry3

            lax.fori_loop(0, ng, grp_body, 0)

        def stream_body(j, carry4):
            c0 = j * 2
            c1 = c0 + 1
            pltpu.make_async_copy(dummy, cbuf.at[0], s0).wait()
            process_chunk(c0, 0)

            @pl.when(c0 + 2 < n_ch)
            def _():
                issue_chunk(c0 + 2, 0, s0)

            pltpu.make_async_copy(dummy, cbuf.at[1], s1).wait()
            process_chunk(c1, 1)

            @pl.when(c1 + 2 < n_ch)
            def _():
                issue_chunk(c1 + 2, 1, s1)

            return carry4

        lax.fori_loop(0, n_ch // 2, stream_body, 0)
        pltpu.make_async_copy(trash, rowbuf, sr).wait()

    return gather_kernel


def kernel(x, embed_weight):
    (B,) = x.shape
    V, D = embed_weight.shape
    tab_t = embed_weight.T
    idx = x.astype(jnp.int32)
    out = _make_gather(B, D, V)(tab_t, idx)
    return out[None]


# speedup vs baseline: 1.2803x; 1.2356x over previous
import functools

import jax
import jax.numpy as jnp
from jax import lax
from jax.experimental import pallas as pl
from jax.experimental.pallas import tpu as pltpu
from jax.experimental.pallas import tpu_sc as plsc

NUM_CORES = 2
NUM_SUBCORES = 16
NUM_WORKERS = NUM_CORES * NUM_SUBCORES


@functools.lru_cache(maxsize=None)
def _make_gather(B, D, sub):
    b_per_w = B // NUM_WORKERS
    mesh = plsc.VectorSubcoreMesh(core_axis_name="c", subcore_axis_name="s")

    @functools.partial(
        pl.kernel,
        mesh=mesh,
        out_type=jax.ShapeDtypeStruct((B, D), jnp.float32),
        scratch_types=[
            pltpu.VMEM((b_per_w,), jnp.int32),
            pltpu.VMEM((b_per_w, D), jnp.float32),
            pltpu.HBM((b_per_w, D), jnp.float32),
            pltpu.SemaphoreType.DMA,
        ],
        compiler_params=pltpu.CompilerParams(
            use_tc_tiling_on_sc=True, needs_layout_passes=False
        ),
    )
    def gather_kernel(tab_hbm, idx_hbm, out_hbm, idx_v, stage, dummy, sem):
        wid = lax.axis_index("s") * NUM_CORES + lax.axis_index("c")
        base = wid * b_per_w
        pltpu.sync_copy(idx_hbm.at[wid], idx_v)

        def group_body(g, carry):
            vec = idx_v[pl.ds(g * 16, 16)]
            t_vec = lax.shift_right_logical(vec, 3)
            s_vec = jnp.bitwise_and(vec, sub - 1)
            for l in range(16):
                pltpu.async_copy(
                    tab_hbm.at[t_vec[l], s_vec[l]],
                    stage.at[g * 16 + l],
                    sem,
                )
            return carry

        lax.fori_loop(0, b_per_w // 16, group_body, 0)
        pltpu.make_async_copy(dummy, stage, sem).wait()
        pltpu.sync_copy(stage, out_hbm.at[pl.ds(base, b_per_w)])

    return gather_kernel


def kernel(x, embed_weight):
    (B,) = x.shape
    V, D = embed_weight.shape
    sub = 8
    tab3 = embed_weight.reshape(V // sub, sub, D)
    idx = x.astype(jnp.int32).reshape(NUM_WORKERS, B // NUM_WORKERS)
    out = _make_gather(B, D, sub)(tab3, idx)
    return out[None]
